# baseline (speedup 1.0000x reference)
import jax
import jax.numpy as jnp
from jax import lax
from jax.experimental import pallas as pl
from jax.experimental.pallas import tpu as pltpu
from jax.experimental.pallas import tpu_sc as plsc

B = 4096
G = 100
OD = 16
NW = 32
BT = B // NW
NCHUNK = BT // 16
ROW = G * OD
OUT_HALF = 16 * ROW


def _permute(x, idx):
    return lax.gather(
        x,
        idx[:, None],
        lax.GatherDimensionNumbers(
            offset_dims=(), collapsed_slice_dims=(0,), start_index_map=(0,)),
        (1,),
        mode=lax.GatherScatterMode.PROMISE_IN_BOUNDS,
    )


def _body(x_hbm, tab_hbm, out_hbm, a_buf, tab_buf, out_buf, sem0, sem1):
    wid = lax.axis_index("s") * 2 + lax.axis_index("c")
    pltpu.sync_copy(x_hbm.at[pl.ds(wid * (BT * 3 * G), BT * 3 * G)], a_buf)
    pltpu.sync_copy(tab_hbm, tab_buf)

    iota = lax.iota(jnp.int32, 16)
    iota300 = iota * (3 * G)
    row_scatter = iota * ROW

    def do_chunk(c, par, sem):
        cbase = c * 16 * (3 * G)

        def g_loop(_):
            def g_body(g, carry):
                xb = iota300 + (cbase + g * 3)
                va0 = plsc.load_gather(a_buf, [xb])
                va1 = plsc.load_gather(a_buf, [xb + 1])
                va2 = plsc.load_gather(a_buf, [xb + 2])
                vmin = jnp.minimum(jnp.minimum(va0, va1), va2)
                vmax = jnp.maximum(jnp.maximum(va0, va1), va2)
                vmid = jnp.maximum(jnp.minimum(va0, va1),
                                   jnp.minimum(jnp.maximum(va0, va1), va2))
                c0 = vmin
                c1 = vmid - vmin
                c2 = vmax - vmid
                pmin = jnp.where(va0 == vmin, jnp.int32(1),
                                 jnp.where(va1 == vmin, jnp.int32(2),
                                           jnp.int32(4)))
                pmax = jnp.where(va2 == vmax, jnp.int32(4),
                                 jnp.where(va1 == vmax, jnp.int32(2),
                                           jnp.int32(1)))
                gbase = g * 136
                idx_mid = gbase + 119 - pmin * 17
                idx_max = gbase + pmax * 17
                row7 = tab_buf[pl.ds(gbase + 119, 16)]
                sc_base = row_scatter + (par + g * OD)

                @plsc.parallel_loop(0, 16, unroll=4)
                def l_body(l):
                    lrot = (iota + l) & 15
                    s7 = _permute(row7, lrot)
                    smid = plsc.load_gather(tab_buf, [idx_mid + lrot])
                    smax = plsc.load_gather(tab_buf, [idx_max + lrot])
                    o = c0 * s7 + c1 * smid + c2 * smax
                    plsc.store_scatter(out_buf, [sc_base + lrot], o)

                return carry

            lax.fori_loop(0, G, g_body, 0)

        g_loop(None)
        b0 = wid * BT + c * 16
        pltpu.async_copy(
            out_buf.at[pl.ds(par, OUT_HALF)],
            out_hbm.at[pl.ds(b0 * ROW, OUT_HALF)],
            sem,
        )

    def wait_half(c, par, sem):
        b0 = wid * BT + c * 16
        pltpu.make_async_copy(
            out_buf.at[pl.ds(par, OUT_HALF)],
            out_hbm.at[pl.ds(b0 * ROW, OUT_HALF)],
            sem,
        ).wait()

    def pair_body(cc, carry):
        c_even = cc * 2
        c_odd = cc * 2 + 1

        @pl.when(cc >= 1)
        def _w0():
            wait_half(c_even - 2, 0, sem0)

        do_chunk(c_even, 0, sem0)

        @pl.when(cc >= 1)
        def _w1():
            wait_half(c_odd - 2, OUT_HALF, sem1)

        do_chunk(c_odd, OUT_HALF, sem1)
        return carry

    lax.fori_loop(0, NCHUNK // 2, pair_body, 0)
    wait_half(NCHUNK - 2, 0, sem0)
    wait_half(NCHUNK - 1, OUT_HALF, sem1)


@jax.jit
def kernel(X, params):
    xb = X.reshape(B * 3 * G)
    tab = jnp.pad(params.reshape(G * 8, OD), ((0, 0), (0, 1))).reshape(G * 8 * 17)
    run = pl.kernel(
        _body,
        out_type=jax.ShapeDtypeStruct((B * ROW,), jnp.float32),
        mesh=plsc.VectorSubcoreMesh(core_axis_name="c", subcore_axis_name="s"),
        compiler_params=pltpu.CompilerParams(needs_layout_passes=False),
        scratch_types=[
            pltpu.VMEM((BT * 3 * G,), jnp.float32),
            pltpu.VMEM((G * 8 * 17,), jnp.float32),
            pltpu.VMEM((2 * OUT_HALF,), jnp.float32),
            pltpu.SemaphoreType.DMA,
            pltpu.SemaphoreType.DMA,
        ],
    )
    out = run(xb, tab)
    return out.reshape(B, ROW)


# speedup vs baseline: 1.2042x; 1.2042x over previous
import jax
import jax.numpy as jnp
from jax import lax
from jax.experimental import pallas as pl
from jax.experimental.pallas import tpu as pltpu
from jax.experimental.pallas import tpu_sc as plsc

B = 4096
G = 100
OD = 16
NW = 32
BT = B // NW
NCHUNK = BT // 16
ROW = G * OD
OUT_HALF = 16 * ROW


def _permute(x, idx):
    return lax.gather(
        x,
        idx[:, None],
        lax.GatherDimensionNumbers(
            offset_dims=(), collapsed_slice_dims=(0,), start_index_map=(0,)),
        (1,),
        mode=lax.GatherScatterMode.PROMISE_IN_BOUNDS,
    )


def _body(x_hbm, tab_hbm, out_hbm, a_buf, tab_buf, out_buf, sem0, sem1):
    wid = lax.axis_index("s") * 2 + lax.axis_index("c")
    pltpu.sync_copy(x_hbm.at[pl.ds(wid * (BT * 3 * G), BT * 3 * G)], a_buf)
    pltpu.sync_copy(tab_hbm, tab_buf)

    iota = lax.iota(jnp.int32, 16)
    iota300 = iota * (3 * G)

    def do_chunk(c, par, sem):
        cbase = c * 16 * (3 * G)

        def g_loop(_):
            def g_body(g, carry):
                xb = iota300 + (cbase + g * 3)
                va0 = plsc.load_gather(a_buf, [xb])
                va1 = plsc.load_gather(a_buf, [xb + 1])
                va2 = plsc.load_gather(a_buf, [xb + 2])
                vmin = jnp.minimum(jnp.minimum(va0, va1), va2)
                vmax = jnp.maximum(jnp.maximum(va0, va1), va2)
                vmid = jnp.maximum(jnp.minimum(va0, va1),
                                   jnp.minimum(jnp.maximum(va0, va1), va2))
                c0 = vmin
                c1 = vmid - vmin
                c2 = vmax - vmid
                pmin = jnp.where(va0 == vmin, jnp.int32(1),
                                 jnp.where(va1 == vmin, jnp.int32(2),
                                           jnp.int32(4)))
                pmax = jnp.where(va2 == vmax, jnp.int32(4),
                                 jnp.where(va1 == vmax, jnp.int32(2),
                                           jnp.int32(1)))
                gbase = g * 136
                idx_mid = gbase + 119 - pmin * 17
                idx_max = gbase + pmax * 17
                row7 = tab_buf[pl.ds(gbase + 119, 16)]
                jrow = iota + par
                col_base = g * OD

                @plsc.parallel_loop(0, 16, unroll=4)
                def l_body(l):
                    lrot = (iota + l) & 15
                    s7 = _permute(row7, lrot)
                    smid = plsc.load_gather(tab_buf, [idx_mid + lrot])
                    smax = plsc.load_gather(tab_buf, [idx_max + lrot])
                    o = c0 * s7 + c1 * smid + c2 * smax
                    plsc.store_scatter(out_buf, [jrow, col_base + lrot], o)

                return carry

            lax.fori_loop(0, G, g_body, 0)

        g_loop(None)
        b0 = wid * BT + c * 16
        pltpu.async_copy(
            out_buf.at[pl.ds(par, 16)],
            out_hbm.at[pl.ds(b0, 16)],
            sem,
        )

    def wait_half(c, par, sem):
        b0 = wid * BT + c * 16
        pltpu.make_async_copy(
            out_buf.at[pl.ds(par, 16)],
            out_hbm.at[pl.ds(b0, 16)],
            sem,
        ).wait()

    def pair_body(cc, carry):
        c_even = cc * 2
        c_odd = cc * 2 + 1

        @pl.when(cc >= 1)
        def _w0():
            wait_half(c_even - 2, 0, sem0)

        do_chunk(c_even, 0, sem0)

        @pl.when(cc >= 1)
        def _w1():
            wait_half(c_odd - 2, 16, sem1)

        do_chunk(c_odd, 16, sem1)
        return carry

    lax.fori_loop(0, NCHUNK // 2, pair_body, 0)
    wait_half(NCHUNK - 2, 0, sem0)
    wait_half(NCHUNK - 1, 16, sem1)


@jax.jit
def kernel(X, params):
    xb = X.reshape(B * 3 * G)
    tab = jnp.pad(params.reshape(G * 8, OD), ((0, 0), (0, 1))).reshape(G * 8 * 17)
    run = pl.kernel(
        _body,
        out_type=jax.ShapeDtypeStruct((B, ROW), jnp.float32),
        mesh=plsc.VectorSubcoreMesh(core_axis_name="c", subcore_axis_name="s"),
        compiler_params=pltpu.CompilerParams(
            needs_layout_passes=False, use_tc_tiling_on_sc=True),
        scratch_types=[
            pltpu.VMEM((BT * 3 * G,), jnp.float32),
            pltpu.VMEM((G * 8 * 17,), jnp.float32),
            pltpu.VMEM((32, ROW), jnp.float32),
            pltpu.SemaphoreType.DMA,
            pltpu.SemaphoreType.DMA,
        ],
    )
    return run(xb, tab)


# speedup vs baseline: 2.2052x; 1.8313x over previous
import jax
import jax.numpy as jnp
from jax import lax
from jax.experimental import pallas as pl
from jax.experimental.pallas import tpu as pltpu
from jax.experimental.pallas import tpu_sc as plsc

B = 4096
G = 100
OD = 16
NW = 32
BT = B // NW
NCHUNK = BT // 16
ROW = G * OD


def _splat(x, l):
    return lax.gather(
        x,
        jnp.zeros((16, 1), jnp.int32) + l,
        lax.GatherDimensionNumbers(
            offset_dims=(), collapsed_slice_dims=(0,), start_index_map=(0,)),
        (1,),
        mode=lax.GatherScatterMode.PROMISE_IN_BOUNDS,
    )


def _body(xt_hbm, tab_hbm, out_hbm, a_buf, tab_buf, out_buf, sem0, sem1):
    wid = lax.axis_index("s") * 2 + lax.axis_index("c")
    pltpu.sync_copy(xt_hbm.at[:, pl.ds(wid * BT, BT)], a_buf)
    pltpu.sync_copy(tab_hbm, tab_buf)

    def do_g(g, par16, sem):
        def c_body(c, carry):
            col = c * 16
            va0 = a_buf[3 * g, pl.ds(col, 16)]
            va1 = a_buf[3 * g + 1, pl.ds(col, 16)]
            va2 = a_buf[3 * g + 2, pl.ds(col, 16)]
            vmin = jnp.minimum(jnp.minimum(va0, va1), va2)
            vmax = jnp.maximum(jnp.maximum(va0, va1), va2)
            vmid = jnp.maximum(jnp.minimum(va0, va1),
                               jnp.minimum(jnp.maximum(va0, va1), va2))
            c0 = vmin
            c1 = vmid - vmin
            c2 = vmax - vmid
            pmin = jnp.where(va0 == vmin, jnp.int32(1),
                             jnp.where(va1 == vmin, jnp.int32(2),
                                       jnp.int32(4)))
            pmax = jnp.where(va2 == vmax, jnp.int32(4),
                             jnp.where(va1 == vmax, jnp.int32(2),
                                       jnp.int32(1)))
            gbase = g * 136
            idx_mid = gbase + 119 - pmin * 17
            idx_max = gbase + pmax * 17
            row7 = tab_buf[pl.ds(gbase + 119, 16)]

            @plsc.parallel_loop(0, 16, unroll=4)
            def l_body(l):
                s7 = _splat(row7, l)
                smid = plsc.load_gather(tab_buf, [idx_mid + l])
                smax = plsc.load_gather(tab_buf, [idx_max + l])
                o = c0 * s7 + c1 * smid + c2 * smax
                out_buf[par16 + l, pl.ds(col, 16)] = o

            return carry

        lax.fori_loop(0, NCHUNK, c_body, 0)
        pltpu.async_copy(
            out_buf.at[pl.ds(par16, 16)],
            out_hbm.at[pl.ds(g * 16, 16), pl.ds(wid * BT, BT)],
            sem,
        )

    def wait_g(g, par16, sem):
        pltpu.make_async_copy(
            out_buf.at[pl.ds(par16, 16)],
            out_hbm.at[pl.ds(g * 16, 16), pl.ds(wid * BT, BT)],
            sem,
        ).wait()

    def pair_body(gg, carry):
        g_even = gg * 2
        g_odd = gg * 2 + 1

        @pl.when(gg >= 1)
        def _w0():
            wait_g(g_even - 2, 0, sem0)

        do_g(g_even, 0, sem0)

        @pl.when(gg >= 1)
        def _w1():
            wait_g(g_odd - 2, 16, sem1)

        do_g(g_odd, 16, sem1)
        return carry

    lax.fori_loop(0, G // 2, pair_body, 0)
    wait_g(G - 2, 0, sem0)
    wait_g(G - 1, 16, sem1)


@jax.jit
def kernel(X, params):
    xt = X.T
    tab = jnp.pad(params.reshape(G * 8, OD), ((0, 0), (0, 1))).reshape(G * 8 * 17)
    run = pl.kernel(
        _body,
        out_type=jax.ShapeDtypeStruct((ROW, B), jnp.float32),
        mesh=plsc.VectorSubcoreMesh(core_axis_name="c", subcore_axis_name="s"),
        compiler_params=pltpu.CompilerParams(
            needs_layout_passes=False, use_tc_tiling_on_sc=True),
        scratch_types=[
            pltpu.VMEM((3 * G, BT), jnp.float32),
            pltpu.VMEM((G * 8 * 17,), jnp.float32),
            pltpu.VMEM((32, BT), jnp.float32),
            pltpu.SemaphoreType.DMA,
            pltpu.SemaphoreType.DMA,
        ],
    )
    out = run(xt, tab)
    return out.T
